# baseline (speedup 1.0000x reference)
import functools

import jax
import jax.numpy as jnp
from jax import lax
from jax.experimental import pallas as pl
from jax.experimental.pallas import tpu as pltpu
from jax.experimental.pallas import tpu_sc as plsc

N_TOKENS = 32768
N_EMBD = 4096
N_CHOICES = 64
HIDDEN = N_EMBD // 2

BT = 256
C = 2
TPC = N_TOKENS // C

NW = 32
TPW = TPC // NW
LANES = 16

_IMIN = -0x80000000


def _logits_body(x_ref, w1_ref, b1_ref, w2_ref, b2_ref, o_ref):
    h = jnp.dot(x_ref[...], w1_ref[...], preferred_element_type=jnp.float32)
    h = jnp.maximum(h + b1_ref[...], 0.0)
    logits = jnp.dot(h, w2_ref[...], preferred_element_type=jnp.float32)
    o_ref[...] = logits + b2_ref[...]


def _router_logits(x, W1, b1, W2, b2, chunk):
    base = chunk * (TPC // BT)
    return pl.pallas_call(
        _logits_body,
        grid=(TPC // BT,),
        in_specs=[
            pl.BlockSpec((BT, N_EMBD), lambda i: (i + base, 0)),
            pl.BlockSpec((N_EMBD, HIDDEN), lambda i: (0, 0)),
            pl.BlockSpec((1, HIDDEN), lambda i: (0, 0)),
            pl.BlockSpec((HIDDEN, N_CHOICES), lambda i: (0, 0)),
            pl.BlockSpec((1, N_CHOICES), lambda i: (0, 0)),
        ],
        out_specs=pl.BlockSpec((BT, N_CHOICES), lambda i: (i, 0)),
        out_shape=jax.ShapeDtypeStruct((TPC, N_CHOICES), jnp.float32),
    )(x, W1, b1, W2, b2)


def _topk_body(logits_hbm, k_hbm, out_hbm, lv, kv, buf):
    wid = lax.axis_index("s") * 2 + lax.axis_index("c")
    base = wid * TPW

    pltpu.sync_copy(k_hbm, kv)
    k_is_1 = kv[...] == 1

    zero16 = jnp.zeros((LANES,), jnp.float32)
    lane = lax.iota(jnp.int32, LANES)
    mask63 = jnp.int32(~63)
    m31 = jnp.int32(0x7FFFFFFF)

    def _unkey(kk):
        ub = kk & mask63
        return plsc.bitcast(ub ^ ((ub >> 31) & m31), jnp.float32)

    pltpu.sync_copy(
        logits_hbm.at[pl.ds(base * N_CHOICES, TPW * N_CHOICES)], lv
    )

    def _group(g, carry):
        rows = g * LANES + lane
        m1 = jnp.full((LANES,), _IMIN, jnp.int32)
        m2 = jnp.full((LANES,), _IMIN, jnp.int32)
        obase = g * (LANES * N_CHOICES)
        for c in range(N_CHOICES):
            lg = plsc.load_gather(lv, [rows * N_CHOICES + jnp.int32(c)])
            b = plsc.bitcast(lg, jnp.int32)
            key = b ^ ((b >> 31) & m31)
            key = (key & mask63) | jnp.int32(63 - c)
            lo = jnp.minimum(m1, key)
            m1 = jnp.maximum(m1, key)
            m2 = jnp.maximum(m2, lo)
            buf[pl.ds(obase + c * LANES, LANES)] = zero16
        i1 = jnp.int32(63) - (m1 & jnp.int32(63))
        i2 = jnp.int32(63) - (m2 & jnp.int32(63))
        p1 = 1.0 / (1.0 + jnp.exp(_unkey(m2) - _unkey(m1)))
        v1 = jnp.where(k_is_1, jnp.float32(1.0), p1)
        v2 = jnp.where(k_is_1, jnp.float32(0.0), 1.0 - p1)
        flat = (g * LANES + lane) * N_CHOICES
        plsc.store_scatter(buf, [flat + i1], v1)
        plsc.store_scatter(buf, [flat + i2], v2)
        return carry

    lax.fori_loop(0, TPW // LANES, _group, 0)

    pltpu.sync_copy(buf, out_hbm.at[pl.ds(base * N_CHOICES, TPW * N_CHOICES)])


_topk_sc = functools.partial(
    pl.kernel,
    out_type=jax.ShapeDtypeStruct((TPC * N_CHOICES,), jnp.float32),
    mesh=plsc.VectorSubcoreMesh(core_axis_name="c", subcore_axis_name="s"),
    compiler_params=pltpu.CompilerParams(needs_layout_passes=False),
    scratch_types=[
        pltpu.VMEM((TPW * N_CHOICES,), jnp.float32),
        pltpu.VMEM((LANES,), jnp.int32),
        pltpu.VMEM((TPW * N_CHOICES,), jnp.float32),
    ],
)(_topk_body)


def kernel(x, W1, b1, W2, b2, k, training):
    b1r = b1.reshape(1, HIDDEN)
    b2r = b2.reshape(1, N_CHOICES)
    k_arr = jnp.full((LANES,), jnp.asarray(k, jnp.int32))
    chunks = []
    for c in range(C):
        logits = _router_logits(x, W1, b1r, W2, b2r, c)
        chunks.append(
            _topk_sc(logits.reshape(TPC * N_CHOICES), k_arr).reshape(
                TPC, N_CHOICES
            )
        )
    return jnp.concatenate(chunks, axis=0)


# speedup vs baseline: 1.0663x; 1.0663x over previous
import functools

import jax
import jax.numpy as jnp
from jax.experimental import pallas as pl
from jax.experimental.pallas import tpu as pltpu

N_TOKENS = 32768
N_EMBD = 4096
N_CHOICES = 64
HIDDEN = N_EMBD // 2

BT = 256


def _router_body(k_ref, x_ref, w1_ref, b1_ref, w2_ref, b2_ref, o_ref):
    h = jnp.dot(x_ref[...], w1_ref[...], preferred_element_type=jnp.float32)
    h = jnp.maximum(h + b1_ref[...], 0.0)
    logits = jnp.dot(h, w2_ref[...], preferred_element_type=jnp.float32)
    logits = logits + b2_ref[...]

    col = jax.lax.broadcasted_iota(jnp.int32, logits.shape, 1)
    b = jax.lax.bitcast_convert_type(logits, jnp.int32)
    key = b ^ ((b >> 31) & jnp.int32(0x7FFFFFFF))
    key = (key & jnp.int32(~63)) | (jnp.int32(63) - col)
    k1 = jnp.max(key, axis=-1, keepdims=True)
    k2 = jnp.max(
        jnp.where(key == k1, jnp.int32(-0x80000000), key), axis=-1, keepdims=True
    )
    i1 = jnp.int32(63) - (k1 & jnp.int32(63))
    i2 = jnp.int32(63) - (k2 & jnp.int32(63))

    def _unkey(kk):
        ub = kk & jnp.int32(~63)
        return jax.lax.bitcast_convert_type(
            ub ^ ((ub >> 31) & jnp.int32(0x7FFFFFFF)), jnp.float32
        )

    p1 = jax.nn.sigmoid(_unkey(k1) - _unkey(k2))
    k_is_1 = k_ref[0] == 1
    v1 = jnp.where(k_is_1, jnp.float32(1.0), p1)
    v2 = jnp.where(k_is_1, jnp.float32(0.0), 1.0 - p1)
    o_ref[...] = jnp.where(col == i1, v1, jnp.where(col == i2, v2, 0.0))


@functools.partial(jax.jit, static_argnames=("interpret",))
def _router(x, W1, b1, W2, b2, k, interpret=False):
    grid = (N_TOKENS // BT,)
    return pl.pallas_call(
        _router_body,
        grid=grid,
        in_specs=[
            pl.BlockSpec(memory_space=pltpu.SMEM),
            pl.BlockSpec((BT, N_EMBD), lambda i: (i, 0)),
            pl.BlockSpec((N_EMBD, HIDDEN), lambda i: (0, 0)),
            pl.BlockSpec((1, HIDDEN), lambda i: (0, 0)),
            pl.BlockSpec((HIDDEN, N_CHOICES), lambda i: (0, 0)),
            pl.BlockSpec((1, N_CHOICES), lambda i: (0, 0)),
        ],
        out_specs=pl.BlockSpec((BT, N_CHOICES), lambda i: (i, 0)),
        out_shape=jax.ShapeDtypeStruct((N_TOKENS, N_CHOICES), jnp.float32),
        interpret=interpret,
    )(k, x, W1, b1, W2, b2)


def kernel(x, W1, b1, W2, b2, k, training):
    k_arr = jnp.asarray(k, jnp.int32).reshape((1,))
    return _router(
        x, W1, b1.reshape(1, HIDDEN), W2, b2.reshape(1, N_CHOICES), k_arr
    )


# speedup vs baseline: 1.0667x; 1.0003x over previous
import jax
import jax.numpy as jnp
from jax.experimental import pallas as pl
from jax.experimental.pallas import tpu as pltpu

N_TOKENS = 32768
N_EMBD = 4096
N_CHOICES = 64
HIDDEN = N_EMBD // 2

BT = 256


def _router_body(k_ref, x_ref, w1_ref, b1_ref, w2_ref, b2_ref, o_ref):
    h = jnp.dot(x_ref[...], w1_ref[...], preferred_element_type=jnp.float32)
    h = jnp.maximum(h + b1_ref[...], 0.0)
    logits = jnp.dot(h, w2_ref[...], preferred_element_type=jnp.float32)
    logits = logits + b2_ref[...]

    col = jax.lax.broadcasted_iota(jnp.int32, logits.shape, 1)
    b = jax.lax.bitcast_convert_type(logits, jnp.int32)
    key = b ^ ((b >> 31) & jnp.int32(0x7FFFFFFF))
    key = (key & jnp.int32(~63)) | (jnp.int32(63) - col)
    k1 = jnp.max(key, axis=-1, keepdims=True)
    k2 = jnp.max(
        jnp.where(key == k1, jnp.int32(-0x80000000), key), axis=-1, keepdims=True
    )
    i1 = jnp.int32(63) - (k1 & jnp.int32(63))
    i2 = jnp.int32(63) - (k2 & jnp.int32(63))

    def _unkey(kk):
        ub = kk & jnp.int32(~63)
        return jax.lax.bitcast_convert_type(
            ub ^ ((ub >> 31) & jnp.int32(0x7FFFFFFF)), jnp.float32
        )

    p1 = jax.nn.sigmoid(_unkey(k1) - _unkey(k2))
    k_is_1 = k_ref[0] == 1
    v1 = jnp.where(k_is_1, jnp.float32(1.0), p1)
    v2 = jnp.where(k_is_1, jnp.float32(0.0), 1.0 - p1)
    o_ref[...] = jnp.where(col == i1, v1, jnp.where(col == i2, v2, 0.0))


@jax.jit
def _router(x, W1, b1, W2, b2, k):
    grid = (N_TOKENS // BT,)
    return pl.pallas_call(
        _router_body,
        grid=grid,
        in_specs=[
            pl.BlockSpec(memory_space=pltpu.SMEM),
            pl.BlockSpec((BT, N_EMBD), lambda i: (i, 0)),
            pl.BlockSpec((N_EMBD, HIDDEN), lambda i: (0, 0)),
            pl.BlockSpec((1, HIDDEN), lambda i: (0, 0)),
            pl.BlockSpec((HIDDEN, N_CHOICES), lambda i: (0, 0)),
            pl.BlockSpec((1, N_CHOICES), lambda i: (0, 0)),
        ],
        out_specs=pl.BlockSpec((BT, N_CHOICES), lambda i: (i, 0)),
        out_shape=jax.ShapeDtypeStruct((N_TOKENS, N_CHOICES), jnp.float32),
    )(k, x, W1, b1, W2, b2)


def kernel(x, W1, b1, W2, b2, k, training):
    k_arr = jnp.asarray(k, jnp.int32).reshape((1,))
    return _router(
        x, W1, b1.reshape(1, HIDDEN), W2, b2.reshape(1, N_CHOICES), k_arr
    )
